# baseline (speedup 1.0000x reference)
import functools

import jax
import jax.numpy as jnp
from jax import lax
from jax.experimental import pallas as pl
from jax.experimental.pallas import tpu as pltpu
from jax.experimental.pallas import tpu_sc as plsc

_SC_NC, _SC_NS = 2, 16
_NW = _SC_NC * _SC_NS
_TPW = 4096 // _NW
_CH = 16

_B, _D, _P, _C = 4096, 2048, 16, 128
_C2 = 2 * _C
_T = 128
_BPAD = _B + _P * _T
_NT = _BPAD // _T
_NTPAD = 64
_TB1 = 1024


def _parent_route_body(x_ref, wp_ref, bp_ref, logits_ref, pos_ref, grp_ref,
                       pc_acc):
    i = pl.program_id(0)
    xb = x_ref[...].astype(jnp.bfloat16)
    logits = lax.dot_general(
        xb, wp_ref[...].astype(jnp.bfloat16), (((1,), (1,)), ((), ())),
        preferred_element_type=jnp.float32) + bp_ref[...]
    logits_ref[...] = logits
    pc_acc[pl.ds(i * _TB1, _TB1), :] = jnp.argmax(
        logits, axis=1, keepdims=True).astype(jnp.int32)

    @pl.when(i == _B // _TB1 - 1)
    def _route():
        _route_math(pc_acc[...], pos_ref, grp_ref)


def _route_math(pcc, pos_ref, grp_ref):
    oh = (pcc == lax.broadcasted_iota(jnp.int32, (_B, _P), 1)
          ).astype(jnp.float32)
    a = oh
    s = 1
    while s < _B:
        a = a + jnp.concatenate(
            [jnp.zeros((s, _P), jnp.float32), a[:-s]], axis=0)
        s *= 2
    ones_col = jnp.ones((_B, 1), jnp.float32)
    counts_col = lax.dot_general(oh, ones_col, (((0,), (0,)), ((), ())))
    capt_col = jnp.floor((counts_col + (_T - 1)) * (1.0 / _T))
    r16 = lax.broadcasted_iota(jnp.int32, (_P, _P), 0)
    c16 = lax.broadcasted_iota(jnp.int32, (_P, _P), 1)
    lstrict = (c16 < r16).astype(jnp.float32)
    tb_col = lax.dot_general(lstrict, capt_col, (((1,), (0,)), ((), ())))
    base_col = tb_col * _T
    rank = jnp.sum(a * oh, axis=1, keepdims=True) - 1.0
    pos_base = lax.dot_general(oh, base_col, (((1,), (0,)), ((), ())))
    pos_ref[...] = (pos_base + rank).astype(jnp.int32)
    used_t = jnp.sum(capt_col)
    t_row = lax.broadcasted_iota(jnp.int32, (_P, _NTPAD), 1
                                 ).astype(jnp.float32)
    cmp = (tb_col <= t_row).astype(jnp.float32)
    grp_row = jnp.sum(cmp, axis=0, keepdims=True) - 1.0
    active = t_row[0:1] < used_t
    grp_ref[...] = jnp.where(active, grp_row, -1.0).astype(jnp.int32)


_NBUF = 3


def _sc_scatter_body(x_hbm, pos_hbm, xs_hbm, idxbuf, buf_a, buf_b, buf_c,
                     sem_i0, sem_i1, sem_i2, sem_o0, sem_o1, sem_o2):
    wid = lax.axis_index("s") * _SC_NC + lax.axis_index("c")
    base = wid * _TPW
    pltpu.sync_copy(pos_hbm.at[pl.ds(base, _TPW)], idxbuf)
    nch = _TPW // _CH
    bufs = (buf_a, buf_b, buf_c)
    isems = (sem_i0, sem_i1, sem_i2)
    osems = (sem_o0, sem_o1, sem_o2)

    def load(c):
        return pltpu.async_copy(
            x_hbm.at[pl.ds(base + c * _CH, _CH)], bufs[c % _NBUF],
            isems[c % _NBUF])

    def scat(c):
        return pltpu.async_copy(
            bufs[c % _NBUF], xs_hbm.at[idxbuf[pl.ds(c * _CH, _CH)]],
            osems[c % _NBUF])

    loads = {}
    scats = {}
    waited = set()
    for c in range(min(_NBUF, nch)):
        loads[c] = load(c)
    for c in range(nch):
        nxt = c - 1 + _NBUF
        if c >= 1 and nxt < nch:
            scats[c - 1].wait()
            waited.add(c - 1)
            loads[nxt] = load(nxt)
        loads[c].wait()
        scats[c] = scat(c)
    for c in range(nch):
        if c not in waited:
            scats[c].wait()


def _sc_gather_body(op_hbm, pos_hbm, c0_hbm, c1_hbm, idxv, rows, sem):
    wid = lax.axis_index("s") * _SC_NC + lax.axis_index("c")
    base = wid * _TPW
    pltpu.sync_copy(pos_hbm.at[pl.ds(base, _TPW)], idxv)
    pltpu.async_copy(op_hbm.at[idxv], rows, sem).wait()
    pltpu.sync_copy(rows.at[:, pl.ds(0, _C)], c0_hbm.at[pl.ds(base, _TPW)])
    pltpu.sync_copy(rows.at[:, pl.ds(_C, _C)], c1_hbm.at[pl.ds(base, _TPW)])


def _wcast_body(w0_ref, w1_ref, w0b_ref, w1b_ref):
    w0b_ref[...] = w0_ref[...].astype(jnp.bfloat16)
    w1b_ref[...] = w1_ref[...].astype(jnp.bfloat16)


def _gmm_body(grp_ref, xs_ref, w0_ref, b0_ref, w1_ref, b1_ref, out_ref):
    t = pl.program_id(0)

    @pl.when(grp_ref[t] >= 0)
    def _():
        xsb = xs_ref[...].astype(jnp.bfloat16)
        out_ref[:, :_C] = lax.dot_general(
            xsb, w0_ref[0], (((1,), (1,)), ((), ())),
            preferred_element_type=jnp.float32) + b0_ref[0, 0]
        out_ref[:, _C:] = lax.dot_general(
            xsb, w1_ref[0], (((1,), (1,)), ((), ())),
            preferred_element_type=jnp.float32) + b1_ref[0, 0]


@jax.jit
def kernel(x, W_parent, b_parent, W_child0, b_child0, W_child1, b_child1,
           device):
    del device

    logits, pos, grp = pl.pallas_call(
        _parent_route_body,
        grid=(_B // _TB1,),
        in_specs=[
            pl.BlockSpec((_TB1, _D), lambda i: (i, 0)),
            pl.BlockSpec((_P, _D), lambda i: (0, 0)),
            pl.BlockSpec((1, _P), lambda i: (0, 0)),
        ],
        out_specs=(
            pl.BlockSpec((_TB1, _P), lambda i: (i, 0)),
            pl.BlockSpec((_B, 1), lambda i: (0, 0)),
            pl.BlockSpec((1, _NTPAD), lambda i: (0, 0)),
        ),
        out_shape=(
            jax.ShapeDtypeStruct((_B, _P), jnp.float32),
            jax.ShapeDtypeStruct((_B, 1), jnp.int32),
            jax.ShapeDtypeStruct((1, _NTPAD), jnp.int32),
        ),
        scratch_shapes=[pltpu.VMEM((_B, 1), jnp.int32)],
    )(x, W_parent, b_parent.reshape(1, _P))
    pos1d = pos.reshape(_B)
    grp1d = grp.reshape(_NTPAD)

    mesh = plsc.VectorSubcoreMesh(
        core_axis_name="c", subcore_axis_name="s",
        num_cores=_SC_NC, num_subcores=_SC_NS)
    xs_pad = pl.kernel(
        _sc_scatter_body,
        out_type=jax.ShapeDtypeStruct((_BPAD, _D), jnp.float32),
        mesh=mesh,
        scratch_types=(
            [pltpu.VMEM((_TPW,), jnp.int32)]
            + [pltpu.VMEM((_CH, _D), jnp.float32)] * _NBUF
            + [pltpu.SemaphoreType.DMA] * (2 * _NBUF)),
    )(x, pos1d)

    grid_spec = pltpu.PrefetchScalarGridSpec(
        num_scalar_prefetch=1,
        grid=(_NT,),
        in_specs=[
            pl.BlockSpec((_T, _D),
                         lambda t, grp: (jnp.where(grp[t] >= 0, t, 0), 0)),
            pl.BlockSpec((1, _C, _D),
                         lambda t, grp: (jnp.maximum(grp[t], 0), 0, 0)),
            pl.BlockSpec((1, 1, _C),
                         lambda t, grp: (jnp.maximum(grp[t], 0), 0, 0)),
            pl.BlockSpec((1, _C, _D),
                         lambda t, grp: (jnp.maximum(grp[t], 0), 0, 0)),
            pl.BlockSpec((1, 1, _C),
                         lambda t, grp: (jnp.maximum(grp[t], 0), 0, 0)),
        ],
        out_specs=pl.BlockSpec((_T, _C2), lambda t, grp: (t, 0)),
    )
    w0b, w1b = pl.pallas_call(
        _wcast_body,
        grid=(_P,),
        in_specs=[pl.BlockSpec((1, _C, _D), lambda p: (p, 0, 0))] * 2,
        out_specs=(pl.BlockSpec((1, _C, _D), lambda p: (p, 0, 0)),) * 2,
        out_shape=(jax.ShapeDtypeStruct((_P, _C, _D), jnp.bfloat16),) * 2,
    )(W_child0, W_child1)

    out_pad = pl.pallas_call(
        _gmm_body,
        grid_spec=grid_spec,
        out_shape=jax.ShapeDtypeStruct((_BPAD, _C2), jnp.float32),
    )(grp1d, xs_pad, w0b, b_child0.reshape(_P, 1, _C),
      w1b, b_child1.reshape(_P, 1, _C))

    child0, child1 = pl.kernel(
        _sc_gather_body,
        out_type=(jax.ShapeDtypeStruct((_B, _C), jnp.float32),
                  jax.ShapeDtypeStruct((_B, _C), jnp.float32)),
        mesh=mesh,
        scratch_types=[
            pltpu.VMEM((_TPW,), jnp.int32),
            pltpu.VMEM((_TPW, _C2), jnp.float32),
            pltpu.SemaphoreType.DMA,
        ],
    )(out_pad, pos1d)
    return logits, child0, child1


# speedup vs baseline: 1.0106x; 1.0106x over previous
import functools

import jax
import jax.numpy as jnp
from jax import lax
from jax.experimental import pallas as pl
from jax.experimental.pallas import tpu as pltpu
from jax.experimental.pallas import tpu_sc as plsc

_SC_NC, _SC_NS = 2, 16
_NW = _SC_NC * _SC_NS
_TPW = 4096 // _NW
_CH = 16

_B, _D, _P, _C = 4096, 2048, 16, 128
_C2 = 2 * _C
_T = 128
_BPAD = _B + _P * _T
_NT = _BPAD // _T
_NTPAD = 64
_TB1 = 1024


def _parent_route_body(x_ref, wp_ref, bp_ref, logits_ref, pos_ref, grp_ref,
                       pc_acc):
    i = pl.program_id(0)
    xb = x_ref[...].astype(jnp.bfloat16)
    logits = lax.dot_general(
        xb, wp_ref[...].astype(jnp.bfloat16), (((1,), (1,)), ((), ())),
        preferred_element_type=jnp.float32) + bp_ref[...]
    logits_ref[...] = logits
    pc_acc[pl.ds(i * _TB1, _TB1), :] = jnp.argmax(
        logits, axis=1, keepdims=True).astype(jnp.int32)

    @pl.when(i == _B // _TB1 - 1)
    def _route():
        _route_math(pc_acc[...], pos_ref, grp_ref)


def _route_math(pcc, pos_ref, grp_ref):
    oh = (pcc == lax.broadcasted_iota(jnp.int32, (_B, _P), 1)
          ).astype(jnp.float32)
    a = oh
    s = 1
    while s < _B:
        a = a + jnp.concatenate(
            [jnp.zeros((s, _P), jnp.float32), a[:-s]], axis=0)
        s *= 2
    ones_col = jnp.ones((_B, 1), jnp.float32)
    counts_col = lax.dot_general(oh, ones_col, (((0,), (0,)), ((), ())))
    capt_col = jnp.floor((counts_col + (_T - 1)) * (1.0 / _T))
    r16 = lax.broadcasted_iota(jnp.int32, (_P, _P), 0)
    c16 = lax.broadcasted_iota(jnp.int32, (_P, _P), 1)
    lstrict = (c16 < r16).astype(jnp.float32)
    tb_col = lax.dot_general(lstrict, capt_col, (((1,), (0,)), ((), ())))
    base_col = tb_col * _T
    rank = jnp.sum(a * oh, axis=1, keepdims=True) - 1.0
    pos_base = lax.dot_general(oh, base_col, (((1,), (0,)), ((), ())))
    pos_ref[...] = (pos_base + rank).astype(jnp.int32)
    used_t = jnp.sum(capt_col)
    t_row = lax.broadcasted_iota(jnp.int32, (_P, _NTPAD), 1
                                 ).astype(jnp.float32)
    cmp = (tb_col <= t_row).astype(jnp.float32)
    grp_row = jnp.sum(cmp, axis=0, keepdims=True) - 1.0
    active = t_row[0:1] < used_t
    grp_ref[...] = jnp.where(active, grp_row, -1.0).astype(jnp.int32)


_NBUF = 3


def _sc_scatter_body(x_hbm, pos_hbm, xs_hbm, idxbuf, buf_a, buf_b, buf_c,
                     sem_i0, sem_i1, sem_i2, sem_o0, sem_o1, sem_o2):
    wid = lax.axis_index("s") * _SC_NC + lax.axis_index("c")
    base = wid * _TPW
    pltpu.sync_copy(pos_hbm.at[pl.ds(base, _TPW)], idxbuf)
    nch = _TPW // _CH
    bufs = (buf_a, buf_b, buf_c)
    isems = (sem_i0, sem_i1, sem_i2)
    osems = (sem_o0, sem_o1, sem_o2)

    def load(c):
        return pltpu.async_copy(
            x_hbm.at[pl.ds(base + c * _CH, _CH)], bufs[c % _NBUF],
            isems[c % _NBUF])

    def scat(c):
        return pltpu.async_copy(
            bufs[c % _NBUF], xs_hbm.at[idxbuf[pl.ds(c * _CH, _CH)]],
            osems[c % _NBUF])

    loads = {}
    scats = {}
    waited = set()
    for c in range(min(_NBUF, nch)):
        loads[c] = load(c)
    for c in range(nch):
        nxt = c - 1 + _NBUF
        if c >= 1 and nxt < nch:
            scats[c - 1].wait()
            waited.add(c - 1)
            loads[nxt] = load(nxt)
        loads[c].wait()
        scats[c] = scat(c)
    for c in range(nch):
        if c not in waited:
            scats[c].wait()


def _sc_gather_body(op_hbm, pos_hbm, c0_hbm, c1_hbm, idxv, rows, sem):
    wid = lax.axis_index("s") * _SC_NC + lax.axis_index("c")
    base = wid * _TPW
    pltpu.sync_copy(pos_hbm.at[pl.ds(base, _TPW)], idxv)
    pltpu.async_copy(op_hbm.at[idxv], rows, sem).wait()
    pltpu.sync_copy(rows.at[:, pl.ds(0, _C)], c0_hbm.at[pl.ds(base, _TPW)])
    pltpu.sync_copy(rows.at[:, pl.ds(_C, _C)], c1_hbm.at[pl.ds(base, _TPW)])


def _gmm_body(grp_ref, xs_ref, w0_ref, b0_ref, w1_ref, b1_ref, out_ref):
    t = pl.program_id(0)

    @pl.when(grp_ref[t] >= 0)
    def _():
        xsb = xs_ref[...].astype(jnp.bfloat16)
        out_ref[:, :_C] = lax.dot_general(
            xsb, w0_ref[0], (((1,), (1,)), ((), ())),
            preferred_element_type=jnp.float32) + b0_ref[0, 0]
        out_ref[:, _C:] = lax.dot_general(
            xsb, w1_ref[0], (((1,), (1,)), ((), ())),
            preferred_element_type=jnp.float32) + b1_ref[0, 0]


@jax.jit
def kernel(x, W_parent, b_parent, W_child0, b_child0, W_child1, b_child1,
           device):
    del device

    logits, pos, grp = pl.pallas_call(
        _parent_route_body,
        grid=(_B // _TB1,),
        in_specs=[
            pl.BlockSpec((_TB1, _D), lambda i: (i, 0)),
            pl.BlockSpec((_P, _D), lambda i: (0, 0)),
            pl.BlockSpec((1, _P), lambda i: (0, 0)),
        ],
        out_specs=(
            pl.BlockSpec((_TB1, _P), lambda i: (i, 0)),
            pl.BlockSpec((_B, 1), lambda i: (0, 0)),
            pl.BlockSpec((1, _NTPAD), lambda i: (0, 0)),
        ),
        out_shape=(
            jax.ShapeDtypeStruct((_B, _P), jnp.float32),
            jax.ShapeDtypeStruct((_B, 1), jnp.int32),
            jax.ShapeDtypeStruct((1, _NTPAD), jnp.int32),
        ),
        scratch_shapes=[pltpu.VMEM((_B, 1), jnp.int32)],
    )(x, W_parent, b_parent.reshape(1, _P))
    pos1d = pos.reshape(_B)
    grp1d = grp.reshape(_NTPAD)

    mesh = plsc.VectorSubcoreMesh(
        core_axis_name="c", subcore_axis_name="s",
        num_cores=_SC_NC, num_subcores=_SC_NS)
    xs_pad = pl.kernel(
        _sc_scatter_body,
        out_type=jax.ShapeDtypeStruct((_BPAD, _D), jnp.float32),
        mesh=mesh,
        scratch_types=(
            [pltpu.VMEM((_TPW,), jnp.int32)]
            + [pltpu.VMEM((_CH, _D), jnp.float32)] * _NBUF
            + [pltpu.SemaphoreType.DMA] * (2 * _NBUF)),
    )(x, pos1d)

    grid_spec = pltpu.PrefetchScalarGridSpec(
        num_scalar_prefetch=1,
        grid=(_NT,),
        in_specs=[
            pl.BlockSpec((_T, _D),
                         lambda t, grp: (jnp.where(grp[t] >= 0, t, 0), 0)),
            pl.BlockSpec((1, _C, _D),
                         lambda t, grp: (jnp.maximum(grp[t], 0), 0, 0)),
            pl.BlockSpec((1, 1, _C),
                         lambda t, grp: (jnp.maximum(grp[t], 0), 0, 0)),
            pl.BlockSpec((1, _C, _D),
                         lambda t, grp: (jnp.maximum(grp[t], 0), 0, 0)),
            pl.BlockSpec((1, 1, _C),
                         lambda t, grp: (jnp.maximum(grp[t], 0), 0, 0)),
        ],
        out_specs=pl.BlockSpec((_T, _C2), lambda t, grp: (t, 0)),
    )
    out_pad = pl.pallas_call(
        _gmm_body,
        grid_spec=grid_spec,
        out_shape=jax.ShapeDtypeStruct((_BPAD, _C2), jnp.float32),
    )(grp1d, xs_pad, W_child0.astype(jnp.bfloat16),
      b_child0.reshape(_P, 1, _C), W_child1.astype(jnp.bfloat16),
      b_child1.reshape(_P, 1, _C))

    child0, child1 = pl.kernel(
        _sc_gather_body,
        out_type=(jax.ShapeDtypeStruct((_B, _C), jnp.float32),
                  jax.ShapeDtypeStruct((_B, _C), jnp.float32)),
        mesh=mesh,
        scratch_types=[
            pltpu.VMEM((_TPW,), jnp.int32),
            pltpu.VMEM((_TPW, _C2), jnp.float32),
            pltpu.SemaphoreType.DMA,
        ],
    )(out_pad, pos1d)
    return logits, child0, child1


# speedup vs baseline: 1.0695x; 1.0583x over previous
import functools

import jax
import jax.numpy as jnp
from jax import lax
from jax.experimental import pallas as pl
from jax.experimental.pallas import tpu as pltpu
from jax.experimental.pallas import tpu_sc as plsc

_SC_NC, _SC_NS = 2, 16
_NW = _SC_NC * _SC_NS
_TPW = 4096 // _NW
_CH = 16

_B, _D, _P, _C = 4096, 2048, 16, 128
_C2 = 2 * _C
_T = 128
_BPAD = _B + _P * _T
_NT = _BPAD // _T
_NTPAD = 64
_TB1 = 2048


def _parent_route_body(x_ref, wp_ref, bp_ref, logits_ref, pos_ref, grp_ref,
                       pc_acc):
    i = pl.program_id(0)
    xb = x_ref[...].astype(jnp.bfloat16)
    logits = lax.dot_general(
        xb, wp_ref[...].astype(jnp.bfloat16), (((1,), (1,)), ((), ())),
        preferred_element_type=jnp.float32) + bp_ref[...]
    logits_ref[...] = logits
    pc_acc[pl.ds(i * _TB1, _TB1), :] = jnp.argmax(
        logits, axis=1, keepdims=True).astype(jnp.int32)

    @pl.when(i == _B // _TB1 - 1)
    def _route():
        _route_math(pc_acc[...], pos_ref, grp_ref)


def _route_math(pcc, pos_ref, grp_ref):
    oh = (pcc == lax.broadcasted_iota(jnp.int32, (_B, _P), 1)
          ).astype(jnp.float32)
    a = oh
    s = 1
    while s < _B:
        a = a + jnp.concatenate(
            [jnp.zeros((s, _P), jnp.float32), a[:-s]], axis=0)
        s *= 2
    ones_col = jnp.ones((_B, 1), jnp.float32)
    counts_col = lax.dot_general(oh, ones_col, (((0,), (0,)), ((), ())))
    capt_col = jnp.floor((counts_col + (_T - 1)) * (1.0 / _T))
    r16 = lax.broadcasted_iota(jnp.int32, (_P, _P), 0)
    c16 = lax.broadcasted_iota(jnp.int32, (_P, _P), 1)
    lstrict = (c16 < r16).astype(jnp.float32)
    tb_col = lax.dot_general(lstrict, capt_col, (((1,), (0,)), ((), ())))
    base_col = tb_col * _T
    rank = jnp.sum(a * oh, axis=1, keepdims=True) - 1.0
    pos_base = lax.dot_general(oh, base_col, (((1,), (0,)), ((), ())))
    pos_ref[...] = (pos_base + rank).astype(jnp.int32)
    used_t = jnp.sum(capt_col)
    t_row = lax.broadcasted_iota(jnp.int32, (_P, _NTPAD), 1
                                 ).astype(jnp.float32)
    cmp = (tb_col <= t_row).astype(jnp.float32)
    grp_row = jnp.sum(cmp, axis=0, keepdims=True) - 1.0
    active = t_row[0:1] < used_t
    grp_ref[...] = jnp.where(active, grp_row, -1.0).astype(jnp.int32)


_NBUF = 3


def _sc_scatter_body(x_hbm, pos_hbm, xs_hbm, idxbuf, buf_a, buf_b, buf_c,
                     sem_i0, sem_i1, sem_i2, sem_o0, sem_o1, sem_o2):
    wid = lax.axis_index("s") * _SC_NC + lax.axis_index("c")
    base = wid * _TPW
    pltpu.sync_copy(pos_hbm.at[pl.ds(base, _TPW)], idxbuf)
    nch = _TPW // _CH
    bufs = (buf_a, buf_b, buf_c)
    isems = (sem_i0, sem_i1, sem_i2)
    osems = (sem_o0, sem_o1, sem_o2)

    def load(c):
        return pltpu.async_copy(
            x_hbm.at[pl.ds(base + c * _CH, _CH)], bufs[c % _NBUF],
            isems[c % _NBUF])

    def scat(c):
        return pltpu.async_copy(
            bufs[c % _NBUF], xs_hbm.at[idxbuf[pl.ds(c * _CH, _CH)]],
            osems[c % _NBUF])

    loads = {}
    scats = {}
    waited = set()
    for c in range(min(_NBUF, nch)):
        loads[c] = load(c)
    for c in range(nch):
        nxt = c - 1 + _NBUF
        if c >= 1 and nxt < nch:
            scats[c - 1].wait()
            waited.add(c - 1)
            loads[nxt] = load(nxt)
        loads[c].wait()
        scats[c] = scat(c)
    for c in range(nch):
        if c not in waited:
            scats[c].wait()


def _sc_gather_body(op_hbm, pos_hbm, c0_hbm, c1_hbm, idxv, rows, sem):
    wid = lax.axis_index("s") * _SC_NC + lax.axis_index("c")
    base = wid * _TPW
    pltpu.sync_copy(pos_hbm.at[pl.ds(base, _TPW)], idxv)
    pltpu.async_copy(op_hbm.at[idxv], rows, sem).wait()
    pltpu.sync_copy(rows.at[:, pl.ds(0, _C)], c0_hbm.at[pl.ds(base, _TPW)])
    pltpu.sync_copy(rows.at[:, pl.ds(_C, _C)], c1_hbm.at[pl.ds(base, _TPW)])


def _gmm_body(grp_ref, xs_ref, w0_ref, b0_ref, w1_ref, b1_ref, out_ref):
    t = pl.program_id(0)

    @pl.when(grp_ref[t] >= 0)
    def _():
        xsb = xs_ref[...].astype(jnp.bfloat16)
        out_ref[:, :_C] = lax.dot_general(
            xsb, w0_ref[0].astype(jnp.bfloat16), (((1,), (1,)), ((), ())),
            preferred_element_type=jnp.float32) + b0_ref[0, 0]
        out_ref[:, _C:] = lax.dot_general(
            xsb, w1_ref[0].astype(jnp.bfloat16), (((1,), (1,)), ((), ())),
            preferred_element_type=jnp.float32) + b1_ref[0, 0]


@jax.jit
def kernel(x, W_parent, b_parent, W_child0, b_child0, W_child1, b_child1,
           device):
    del device

    logits, pos, grp = pl.pallas_call(
        _parent_route_body,
        grid=(_B // _TB1,),
        in_specs=[
            pl.BlockSpec((_TB1, _D), lambda i: (i, 0)),
            pl.BlockSpec((_P, _D), lambda i: (0, 0)),
            pl.BlockSpec((1, _P), lambda i: (0, 0)),
        ],
        out_specs=(
            pl.BlockSpec((_TB1, _P), lambda i: (i, 0)),
            pl.BlockSpec((_B, 1), lambda i: (0, 0)),
            pl.BlockSpec((1, _NTPAD), lambda i: (0, 0)),
        ),
        out_shape=(
            jax.ShapeDtypeStruct((_B, _P), jnp.float32),
            jax.ShapeDtypeStruct((_B, 1), jnp.int32),
            jax.ShapeDtypeStruct((1, _NTPAD), jnp.int32),
        ),
        scratch_shapes=[pltpu.VMEM((_B, 1), jnp.int32)],
    )(x, W_parent, b_parent.reshape(1, _P))
    pos1d = pos.reshape(_B)
    grp1d = grp.reshape(_NTPAD)

    mesh = plsc.VectorSubcoreMesh(
        core_axis_name="c", subcore_axis_name="s",
        num_cores=_SC_NC, num_subcores=_SC_NS)
    xs_pad = pl.kernel(
        _sc_scatter_body,
        out_type=jax.ShapeDtypeStruct((_BPAD, _D), jnp.float32),
        mesh=mesh,
        scratch_types=(
            [pltpu.VMEM((_TPW,), jnp.int32)]
            + [pltpu.VMEM((_CH, _D), jnp.float32)] * _NBUF
            + [pltpu.SemaphoreType.DMA] * (2 * _NBUF)),
    )(x, pos1d)

    grid_spec = pltpu.PrefetchScalarGridSpec(
        num_scalar_prefetch=1,
        grid=(_NT,),
        in_specs=[
            pl.BlockSpec((_T, _D),
                         lambda t, grp: (jnp.where(grp[t] >= 0, t, 0), 0)),
            pl.BlockSpec((1, _C, _D),
                         lambda t, grp: (jnp.maximum(grp[t], 0), 0, 0)),
            pl.BlockSpec((1, 1, _C),
                         lambda t, grp: (jnp.maximum(grp[t], 0), 0, 0)),
            pl.BlockSpec((1, _C, _D),
                         lambda t, grp: (jnp.maximum(grp[t], 0), 0, 0)),
            pl.BlockSpec((1, 1, _C),
                         lambda t, grp: (jnp.maximum(grp[t], 0), 0, 0)),
        ],
        out_specs=pl.BlockSpec((_T, _C2), lambda t, grp: (t, 0)),
    )
    out_pad = pl.pallas_call(
        _gmm_body,
        grid_spec=grid_spec,
        out_shape=jax.ShapeDtypeStruct((_BPAD, _C2), jnp.float32),
    )(grp1d, xs_pad, W_child0, b_child0.reshape(_P, 1, _C),
      W_child1, b_child1.reshape(_P, 1, _C))

    child0, child1 = pl.kernel(
        _sc_gather_body,
        out_type=(jax.ShapeDtypeStruct((_B, _C), jnp.float32),
                  jax.ShapeDtypeStruct((_B, _C), jnp.float32)),
        mesh=mesh,
        scratch_types=[
            pltpu.VMEM((_TPW,), jnp.int32),
            pltpu.VMEM((_TPW, _C2), jnp.float32),
            pltpu.SemaphoreType.DMA,
        ],
    )(out_pad, pos1d)
    return logits, child0, child1


# speedup vs baseline: 1.0824x; 1.0120x over previous
import functools

import jax
import jax.numpy as jnp
from jax import lax
from jax.experimental import pallas as pl
from jax.experimental.pallas import tpu as pltpu
from jax.experimental.pallas import tpu_sc as plsc

_SC_NC, _SC_NS = 2, 16
_NW = _SC_NC * _SC_NS
_TPW = 4096 // _NW
_CH = 16

_B, _D, _P, _C = 4096, 2048, 16, 128
_C2 = 2 * _C
_T = 128
_BPAD = _B + _P * _T
_NT = _BPAD // _T
_NTPAD = 64
_TB1 = 1024


def _parent_route_body(x_ref, wp_ref, bp_ref, logits_ref, pos_ref, grp_ref,
                       pc_acc):
    i = pl.program_id(0)
    xb = x_ref[...].astype(jnp.bfloat16)
    logits = lax.dot_general(
        xb, wp_ref[...].astype(jnp.bfloat16), (((1,), (1,)), ((), ())),
        preferred_element_type=jnp.float32) + bp_ref[...]
    logits_ref[...] = logits
    pc_acc[pl.ds(i * _TB1, _TB1), :] = jnp.argmax(
        logits, axis=1, keepdims=True).astype(jnp.int32)

    @pl.when(i == _B // _TB1 - 1)
    def _route():
        _route_math(pc_acc[...], pos_ref, grp_ref)


def _route_math(pcc, pos_ref, grp_ref):
    oh = (pcc == lax.broadcasted_iota(jnp.int32, (_B, _P), 1)
          ).astype(jnp.float32)
    a = oh
    s = 1
    while s < _B:
        a = a + jnp.concatenate(
            [jnp.zeros((s, _P), jnp.float32), a[:-s]], axis=0)
        s *= 2
    ones_col = jnp.ones((_B, 1), jnp.float32)
    counts_col = lax.dot_general(oh, ones_col, (((0,), (0,)), ((), ())))
    capt_col = jnp.floor((counts_col + (_T - 1)) * (1.0 / _T))
    r16 = lax.broadcasted_iota(jnp.int32, (_P, _P), 0)
    c16 = lax.broadcasted_iota(jnp.int32, (_P, _P), 1)
    lstrict = (c16 < r16).astype(jnp.float32)
    tb_col = lax.dot_general(lstrict, capt_col, (((1,), (0,)), ((), ())))
    base_col = tb_col * _T
    rank = jnp.sum(a * oh, axis=1, keepdims=True) - 1.0
    pos_base = lax.dot_general(oh, base_col, (((1,), (0,)), ((), ())))
    pos_ref[...] = (pos_base + rank).astype(jnp.int32)
    used_t = jnp.sum(capt_col)
    t_row = lax.broadcasted_iota(jnp.int32, (_P, _NTPAD), 1
                                 ).astype(jnp.float32)
    cmp = (tb_col <= t_row).astype(jnp.float32)
    grp_row = jnp.sum(cmp, axis=0, keepdims=True) - 1.0
    active = t_row[0:1] < used_t
    grp_ref[...] = jnp.where(active, grp_row, -1.0).astype(jnp.int32)


_NBUF = 3


def _sc_scatter_body(x_hbm, pos_hbm, xs_hbm, idxbuf, buf_a, buf_b, buf_c,
                     sem_i0, sem_i1, sem_i2, sem_o0, sem_o1, sem_o2):
    wid = lax.axis_index("s") * _SC_NC + lax.axis_index("c")
    base = wid * _TPW
    pltpu.sync_copy(pos_hbm.at[pl.ds(base, _TPW)], idxbuf)
    nch = _TPW // _CH
    bufs = (buf_a, buf_b, buf_c)
    isems = (sem_i0, sem_i1, sem_i2)
    osems = (sem_o0, sem_o1, sem_o2)

    def load(c):
        return pltpu.async_copy(
            x_hbm.at[pl.ds(base + c * _CH, _CH)], bufs[c % _NBUF],
            isems[c % _NBUF])

    def scat(c):
        return pltpu.async_copy(
            bufs[c % _NBUF], xs_hbm.at[idxbuf[pl.ds(c * _CH, _CH)]],
            osems[c % _NBUF])

    loads = {}
    scats = {}
    waited = set()
    for c in range(min(_NBUF, nch)):
        loads[c] = load(c)
    for c in range(nch):
        nxt = c - 1 + _NBUF
        if c >= 1 and nxt < nch:
            scats[c - 1].wait()
            waited.add(c - 1)
            loads[nxt] = load(nxt)
        loads[c].wait()
        scats[c] = scat(c)
    for c in range(nch):
        if c not in waited:
            scats[c].wait()


def _sc_gather_body(op_hbm, pos_hbm, c0_hbm, c1_hbm, idxv, rows, sem):
    wid = lax.axis_index("s") * _SC_NC + lax.axis_index("c")
    base = wid * _TPW
    pltpu.sync_copy(pos_hbm.at[pl.ds(base, _TPW)], idxv)
    pltpu.async_copy(op_hbm.at[idxv], rows, sem).wait()
    pltpu.sync_copy(rows.at[:, pl.ds(0, _C)], c0_hbm.at[pl.ds(base, _TPW)])
    pltpu.sync_copy(rows.at[:, pl.ds(_C, _C)], c1_hbm.at[pl.ds(base, _TPW)])


def _gmm_body(grp_ref, xs_ref, w0_ref, b0_ref, w1_ref, b1_ref, out_ref):
    t = pl.program_id(0)

    @pl.when(grp_ref[t] >= 0)
    def _():
        xsb = xs_ref[...].astype(jnp.bfloat16)
        out_ref[:, :_C] = lax.dot_general(
            xsb, w0_ref[0].astype(jnp.bfloat16), (((1,), (1,)), ((), ())),
            preferred_element_type=jnp.float32) + b0_ref[0, 0]
        out_ref[:, _C:] = lax.dot_general(
            xsb, w1_ref[0].astype(jnp.bfloat16), (((1,), (1,)), ((), ())),
            preferred_element_type=jnp.float32) + b1_ref[0, 0]


@jax.jit
def kernel(x, W_parent, b_parent, W_child0, b_child0, W_child1, b_child1,
           device):
    del device

    logits, pos, grp = pl.pallas_call(
        _parent_route_body,
        grid=(_B // _TB1,),
        in_specs=[
            pl.BlockSpec((_TB1, _D), lambda i: (i, 0)),
            pl.BlockSpec((_P, _D), lambda i: (0, 0)),
            pl.BlockSpec((1, _P), lambda i: (0, 0)),
        ],
        out_specs=(
            pl.BlockSpec((_TB1, _P), lambda i: (i, 0)),
            pl.BlockSpec((_B, 1), lambda i: (0, 0)),
            pl.BlockSpec((1, _NTPAD), lambda i: (0, 0)),
        ),
        out_shape=(
            jax.ShapeDtypeStruct((_B, _P), jnp.float32),
            jax.ShapeDtypeStruct((_B, 1), jnp.int32),
            jax.ShapeDtypeStruct((1, _NTPAD), jnp.int32),
        ),
        scratch_shapes=[pltpu.VMEM((_B, 1), jnp.int32)],
    )(x, W_parent, b_parent.reshape(1, _P))
    pos1d = pos.reshape(_B)
    grp1d = grp.reshape(_NTPAD)

    mesh = plsc.VectorSubcoreMesh(
        core_axis_name="c", subcore_axis_name="s",
        num_cores=_SC_NC, num_subcores=_SC_NS)
    xs_pad = pl.kernel(
        _sc_scatter_body,
        out_type=jax.ShapeDtypeStruct((_BPAD, _D), jnp.float32),
        mesh=mesh,
        scratch_types=(
            [pltpu.VMEM((_TPW,), jnp.int32)]
            + [pltpu.VMEM((_CH, _D), jnp.float32)] * _NBUF
            + [pltpu.SemaphoreType.DMA] * (2 * _NBUF)),
    )(x, pos1d)

    grid_spec = pltpu.PrefetchScalarGridSpec(
        num_scalar_prefetch=1,
        grid=(_NT,),
        in_specs=[
            pl.BlockSpec((_T, _D),
                         lambda t, grp: (jnp.where(grp[t] >= 0, t, 0), 0)),
            pl.BlockSpec((1, _C, _D),
                         lambda t, grp: (jnp.maximum(grp[t], 0), 0, 0)),
            pl.BlockSpec((1, 1, _C),
                         lambda t, grp: (jnp.maximum(grp[t], 0), 0, 0)),
            pl.BlockSpec((1, _C, _D),
                         lambda t, grp: (jnp.maximum(grp[t], 0), 0, 0)),
            pl.BlockSpec((1, 1, _C),
                         lambda t, grp: (jnp.maximum(grp[t], 0), 0, 0)),
        ],
        out_specs=pl.BlockSpec((_T, _C2), lambda t, grp: (t, 0)),
    )
    out_pad = pl.pallas_call(
        _gmm_body,
        grid_spec=grid_spec,
        out_shape=jax.ShapeDtypeStruct((_BPAD, _C2), jnp.float32),
    )(grp1d, xs_pad, W_child0, b_child0.reshape(_P, 1, _C),
      W_child1, b_child1.reshape(_P, 1, _C))

    child0, child1 = pl.kernel(
        _sc_gather_body,
        out_type=(jax.ShapeDtypeStruct((_B, _C), jnp.float32),
                  jax.ShapeDtypeStruct((_B, _C), jnp.float32)),
        mesh=mesh,
        scratch_types=[
            pltpu.VMEM((_TPW,), jnp.int32),
            pltpu.VMEM((_TPW, _C2), jnp.float32),
            pltpu.SemaphoreType.DMA,
        ],
    )(out_pad, pos1d)
    return logits, child0, child1
